# Initial kernel scaffold; baseline (speedup 1.0000x reference)
import jax
import jax.numpy as jnp
from jax.experimental import pallas as pl


def kernel(node_fea, edge_index, edge_fea, W, b):
    raise NotImplementedError("write your pallas kernel here")


# speedup vs baseline: 3.3765x; 3.3765x over previous
import jax
import jax.numpy as jnp
from jax import lax
from jax.experimental import pallas as pl
from jax.experimental.pallas import tpu as pltpu
from jax.experimental.pallas import tpu_sc as plsc

N = 10000
E = 320000
D = 128
G = 128
CHUNK = 80
NUM_CHUNKS = E // CHUNK
NC, NS = 2, 16
NW = NC * NS
TILE_CHUNKS = NUM_CHUNKS // NW
RCHUNKS = N // CHUNK
RC_PER_TILE = -(-RCHUNKS // NS)


def _sc_h_body(node_hbm, src_hbm, dst_hbm, edge_hbm, zrow_hbm, h_out,
               src_idx, dst_idx, gath_v, edge_v, h_sh, sem1, sem2):
  cid = lax.axis_index("c")
  sid = lax.axis_index("s")
  wid = sid * NC + cid

  pltpu.sync_copy(zrow_hbm, gath_v)

  @pl.loop(0, RC_PER_TILE)
  def _(j):
    r = sid * RC_PER_TILE + j

    @pl.when(r < RCHUNKS)
    def _():
      pltpu.sync_copy(gath_v, h_sh.at[pl.ds(r * CHUNK, CHUNK)])

  plsc.subcore_barrier()

  base_chunk = wid * TILE_CHUNKS

  @pl.loop(0, TILE_CHUNKS)
  def _(j):
    c = base_chunk + j
    pltpu.sync_copy(src_hbm.at[pl.ds(c * CHUNK, CHUNK)], src_idx)
    pltpu.sync_copy(dst_hbm.at[pl.ds(c * CHUNK, CHUNK)], dst_idx)
    gd = pltpu.async_copy(node_hbm.at[src_idx], gath_v, sem1)
    ed = pltpu.async_copy(edge_hbm.at[pl.ds(c * CHUNK, CHUNK)], edge_v, sem2)
    gd.wait()
    ed.wait()
    pltpu.sync_copy(gath_v, h_sh.at[dst_idx], add=True)
    pltpu.sync_copy(edge_v, h_sh.at[dst_idx], add=True)

  plsc.subcore_barrier()

  @pl.loop(0, RC_PER_TILE)
  def _(j):
    r = sid * RC_PER_TILE + j

    @pl.when(r < RCHUNKS)
    def _():
      pltpu.sync_copy(h_sh.at[pl.ds(r * CHUNK, CHUNK)], gath_v)
      pltpu.sync_copy(gath_v, h_out.at[cid, pl.ds(r * CHUNK, CHUNK)])


def _sc_deg_body(dst_hbm, zdeg_hbm, ones_hbm, deg_out,
                 dst_idx, deg_b, ones_v, deg_sh):
  cid = lax.axis_index("c")
  sid = lax.axis_index("s")
  wid = sid * NC + cid

  pltpu.sync_copy(zdeg_hbm, deg_b)
  pltpu.sync_copy(ones_hbm, ones_v)

  @pl.loop(0, RC_PER_TILE)
  def _(j):
    r = sid * RC_PER_TILE + j

    @pl.when(r < RCHUNKS)
    def _():
      pltpu.sync_copy(deg_b, deg_sh.at[pl.ds(r * CHUNK, CHUNK)])

  plsc.subcore_barrier()

  base_chunk = wid * TILE_CHUNKS

  @pl.loop(0, TILE_CHUNKS)
  def _(j):
    c = base_chunk + j
    pltpu.sync_copy(dst_hbm.at[pl.ds(c * CHUNK, CHUNK)], dst_idx)
    pltpu.sync_copy(ones_v, deg_sh.at[dst_idx], add=True)

  plsc.subcore_barrier()

  @pl.loop(0, RC_PER_TILE)
  def _(j):
    r = sid * RC_PER_TILE + j

    @pl.when(r < RCHUNKS)
    def _():
      pltpu.sync_copy(deg_sh.at[pl.ds(r * CHUNK, CHUNK)], deg_b)
      pltpu.sync_copy(deg_b, deg_out.at[cid, pl.ds(r * CHUNK, CHUNK)])


def _segment_sums(node_fea, src, dst, edge_fea, zrow, zdeg, ones):
  mesh = plsc.VectorSubcoreMesh(core_axis_name="c", subcore_axis_name="s")
  run_h = pl.kernel(
      _sc_h_body,
      out_type=jax.ShapeDtypeStruct((NC, N, D), jnp.float32),
      mesh=mesh,
      scratch_types=[
          pltpu.VMEM((CHUNK,), jnp.int32),
          pltpu.VMEM((CHUNK,), jnp.int32),
          pltpu.VMEM((CHUNK, D), jnp.float32),
          pltpu.VMEM((CHUNK, D), jnp.float32),
          pltpu.VMEM_SHARED((N, D), jnp.float32),
          pltpu.SemaphoreType.DMA,
          pltpu.SemaphoreType.DMA,
      ],
  )
  run_deg = pl.kernel(
      _sc_deg_body,
      out_type=jax.ShapeDtypeStruct((NC, N, G), jnp.float32),
      mesh=mesh,
      scratch_types=[
          pltpu.VMEM((CHUNK,), jnp.int32),
          pltpu.VMEM((CHUNK, G), jnp.float32),
          pltpu.VMEM((CHUNK, G), jnp.float32),
          pltpu.VMEM_SHARED((N, G), jnp.float32),
      ],
  )
  return run_h(node_fea, src, dst, edge_fea, zrow), run_deg(dst, zdeg, ones)


def _tc_body(hp_ref, dp_ref, node_ref, wt_ref, b_ref, out_ref):
  h = hp_ref[0] + hp_ref[1]
  deg = dp_ref[0, :, :1] + dp_ref[1, :, :1]
  rst = jnp.dot(h, wt_ref[...], preferred_element_type=jnp.float32) + b_ref[...]
  scale = lax.rsqrt(jnp.maximum(deg, 1.0))
  out_ref[...] = rst * scale + node_ref[...]


def _finalize(h_part, deg_part, node_fea, Wt, b2):
  R = 2000
  grid = N // R
  return pl.pallas_call(
      _tc_body,
      grid=(grid,),
      in_specs=[
          pl.BlockSpec((NC, R, D), lambda i: (0, i, 0)),
          pl.BlockSpec((NC, R, G), lambda i: (0, i, 0)),
          pl.BlockSpec((R, D), lambda i: (i, 0)),
          pl.BlockSpec((D, D), lambda i: (0, 0)),
          pl.BlockSpec((1, D), lambda i: (0, 0)),
      ],
      out_specs=pl.BlockSpec((R, D), lambda i: (i, 0)),
      out_shape=jax.ShapeDtypeStruct((N, D), jnp.float32),
  )(h_part, deg_part, node_fea, Wt, b2)


@jax.jit
def kernel(node_fea, edge_index, edge_fea, W, b):
  src = edge_index[0]
  dst = edge_index[1]
  zrow = jnp.zeros((CHUNK, D), jnp.float32)
  zdeg = jnp.zeros((CHUNK, G), jnp.float32)
  ones = jnp.ones((CHUNK, G), jnp.float32)
  h_part, deg_part = _segment_sums(node_fea, src, dst, edge_fea, zrow, zdeg,
                                   ones)
  return _finalize(h_part, deg_part, node_fea, W.T, b.reshape(1, D))


# speedup vs baseline: 4.6363x; 1.3731x over previous
import jax
import jax.numpy as jnp
from jax import lax
from jax.experimental import pallas as pl
from jax.experimental.pallas import tpu as pltpu
from jax.experimental.pallas import tpu_sc as plsc

N = 10000
E = 320000
D = 128
G = 128
CHUNK = 80
NUM_CHUNKS = E // CHUNK
NC, NS = 2, 16
NW = NC * NS
TILE_CHUNKS = NUM_CHUNKS // NW
RCHUNKS = N // CHUNK
RC_PER_TILE = -(-RCHUNKS // NS)


def _sc_h_body(node_hbm, src_hbm, dst_hbm, edge_hbm, zrow_hbm, h_out,
               src_idx0, dst_idx0, gath_v0, edge_v0,
               src_idx1, dst_idx1, gath_v1, edge_v1,
               h_sh, semg0, seme0, semg1, seme1):
  cid = lax.axis_index("c")
  sid = lax.axis_index("s")
  wid = sid * NC + cid

  bufs = ((src_idx0, dst_idx0, gath_v0, edge_v0, semg0, seme0),
          (src_idx1, dst_idx1, gath_v1, edge_v1, semg1, seme1))

  pltpu.sync_copy(zrow_hbm, gath_v0)

  @pl.loop(0, RC_PER_TILE)
  def _(j):
    r = sid * RC_PER_TILE + j

    @pl.when(r < RCHUNKS)
    def _():
      pltpu.sync_copy(gath_v0, h_sh.at[pl.ds(r * CHUNK, CHUNK)])

  plsc.subcore_barrier()

  base_chunk = wid * TILE_CHUNKS

  def launch(c, b):
    src_idx, dst_idx, gath_v, edge_v, semg, seme = bufs[b]
    pltpu.sync_copy(src_hbm.at[pl.ds(c * CHUNK, CHUNK)], src_idx)
    pltpu.sync_copy(dst_hbm.at[pl.ds(c * CHUNK, CHUNK)], dst_idx)
    pltpu.async_copy(node_hbm.at[src_idx], gath_v, semg)
    pltpu.async_copy(edge_hbm.at[pl.ds(c * CHUNK, CHUNK)], edge_v, seme)

  def fire(c, b):
    src_idx, dst_idx, gath_v, edge_v, semg, seme = bufs[b]
    pltpu.make_async_copy(node_hbm.at[src_idx], gath_v, semg).wait()
    pltpu.make_async_copy(edge_hbm.at[pl.ds(c * CHUNK, CHUNK)], edge_v,
                          seme).wait()
    pltpu.sync_copy(gath_v, h_sh.at[dst_idx], add=True)
    pltpu.sync_copy(edge_v, h_sh.at[dst_idx], add=True)

  launch(base_chunk, 0)

  @pl.loop(0, (TILE_CHUNKS - 1) // 2)
  def _(j):
    c = base_chunk + 2 * j
    launch(c + 1, 1)
    fire(c, 0)
    launch(c + 2, 0)
    fire(c + 1, 1)

  fire(base_chunk + TILE_CHUNKS - 1, 0)

  plsc.subcore_barrier()

  @pl.loop(0, RC_PER_TILE)
  def _(j):
    r = sid * RC_PER_TILE + j

    @pl.when(r < RCHUNKS)
    def _():
      pltpu.sync_copy(h_sh.at[pl.ds(r * CHUNK, CHUNK)], gath_v0)
      pltpu.sync_copy(gath_v0, h_out.at[cid, pl.ds(r * CHUNK, CHUNK)])


def _sc_deg_body(dst_hbm, zdeg_hbm, ones_hbm, deg_out,
                 dst_idx, deg_b, ones_v, deg_sh):
  cid = lax.axis_index("c")
  sid = lax.axis_index("s")
  wid = sid * NC + cid

  pltpu.sync_copy(zdeg_hbm, deg_b)
  pltpu.sync_copy(ones_hbm, ones_v)

  @pl.loop(0, RC_PER_TILE)
  def _(j):
    r = sid * RC_PER_TILE + j

    @pl.when(r < RCHUNKS)
    def _():
      pltpu.sync_copy(deg_b, deg_sh.at[pl.ds(r * CHUNK, CHUNK)])

  plsc.subcore_barrier()

  base_chunk = wid * TILE_CHUNKS

  @pl.loop(0, TILE_CHUNKS)
  def _(j):
    c = base_chunk + j
    pltpu.sync_copy(dst_hbm.at[pl.ds(c * CHUNK, CHUNK)], dst_idx)
    pltpu.sync_copy(ones_v, deg_sh.at[dst_idx], add=True)

  plsc.subcore_barrier()

  @pl.loop(0, RC_PER_TILE)
  def _(j):
    r = sid * RC_PER_TILE + j

    @pl.when(r < RCHUNKS)
    def _():
      pltpu.sync_copy(deg_sh.at[pl.ds(r * CHUNK, CHUNK)], deg_b)
      pltpu.sync_copy(deg_b, deg_out.at[cid, pl.ds(r * CHUNK, CHUNK)])


def _segment_sums(node_fea, src, dst, edge_fea, zrow, zdeg, ones):
  mesh = plsc.VectorSubcoreMesh(core_axis_name="c", subcore_axis_name="s")
  run_h = pl.kernel(
      _sc_h_body,
      out_type=jax.ShapeDtypeStruct((NC, N, D), jnp.float32),
      mesh=mesh,
      scratch_types=[
          pltpu.VMEM((CHUNK,), jnp.int32),
          pltpu.VMEM((CHUNK,), jnp.int32),
          pltpu.VMEM((CHUNK, D), jnp.float32),
          pltpu.VMEM((CHUNK, D), jnp.float32),
          pltpu.VMEM((CHUNK,), jnp.int32),
          pltpu.VMEM((CHUNK,), jnp.int32),
          pltpu.VMEM((CHUNK, D), jnp.float32),
          pltpu.VMEM((CHUNK, D), jnp.float32),
          pltpu.VMEM_SHARED((N, D), jnp.float32),
          pltpu.SemaphoreType.DMA,
          pltpu.SemaphoreType.DMA,
          pltpu.SemaphoreType.DMA,
          pltpu.SemaphoreType.DMA,
      ],
  )
  run_deg = pl.kernel(
      _sc_deg_body,
      out_type=jax.ShapeDtypeStruct((NC, N, G), jnp.float32),
      mesh=mesh,
      scratch_types=[
          pltpu.VMEM((CHUNK,), jnp.int32),
          pltpu.VMEM((CHUNK, G), jnp.float32),
          pltpu.VMEM((CHUNK, G), jnp.float32),
          pltpu.VMEM_SHARED((N, G), jnp.float32),
      ],
  )
  return run_h(node_fea, src, dst, edge_fea, zrow), run_deg(dst, zdeg, ones)


def _tc_body(hp_ref, dp_ref, node_ref, wt_ref, b_ref, out_ref):
  h = hp_ref[0] + hp_ref[1]
  deg = dp_ref[0, :, :1] + dp_ref[1, :, :1]
  rst = jnp.dot(h, wt_ref[...], preferred_element_type=jnp.float32) + b_ref[...]
  scale = lax.rsqrt(jnp.maximum(deg, 1.0))
  out_ref[...] = rst * scale + node_ref[...]


def _finalize(h_part, deg_part, node_fea, Wt, b2):
  R = 2000
  grid = N // R
  return pl.pallas_call(
      _tc_body,
      grid=(grid,),
      in_specs=[
          pl.BlockSpec((NC, R, D), lambda i: (0, i, 0)),
          pl.BlockSpec((NC, R, G), lambda i: (0, i, 0)),
          pl.BlockSpec((R, D), lambda i: (i, 0)),
          pl.BlockSpec((D, D), lambda i: (0, 0)),
          pl.BlockSpec((1, D), lambda i: (0, 0)),
      ],
      out_specs=pl.BlockSpec((R, D), lambda i: (i, 0)),
      out_shape=jax.ShapeDtypeStruct((N, D), jnp.float32),
  )(h_part, deg_part, node_fea, Wt, b2)


@jax.jit
def kernel(node_fea, edge_index, edge_fea, W, b):
  src = edge_index[0]
  dst = edge_index[1]
  zrow = jnp.zeros((CHUNK, D), jnp.float32)
  zdeg = jnp.zeros((CHUNK, G), jnp.float32)
  ones = jnp.ones((CHUNK, G), jnp.float32)
  h_part, deg_part = _segment_sums(node_fea, src, dst, edge_fea, zrow, zdeg,
                                   ones)
  return _finalize(h_part, deg_part, node_fea, W.T, b.reshape(1, D))
